# Initial kernel scaffold; baseline (speedup 1.0000x reference)
import jax
import jax.numpy as jnp
from jax.experimental import pallas as pl


def kernel(x):
    raise NotImplementedError("write your pallas kernel here")


# speedup vs baseline: 18.6024x; 18.6024x over previous
import functools

import jax
import jax.numpy as jnp
from jax import lax
from jax.experimental import pallas as pl
from jax.experimental.pallas import tpu as pltpu
from jax.experimental.pallas import tpu_sc as plsc

_PATCH = 200
_STRIDE = 150
_MAXP = 1000
_T = 150050
_NC, _NS = 2, 16
_NW = _NC * _NS
_K = 100
_NCHUNK = _MAXP // _K
_INLEN = _K * _STRIDE + 56
_OUTLEN = _K * _PATCH
_VOFF = tuple(16 * j for j in range(12)) + (184,)


def _sc_body(x_hbm, out_hbm, in_v, out_v, sem):
    w = lax.axis_index("s") * _NC + lax.axis_index("c")

    def chunk(ci, _):
        p0 = ci * _K
        pltpu.sync_copy(x_hbm.at[w, pl.ds(_STRIDE * p0, _INLEN)], in_v)

        def patch(k, _):
            src = _STRIDE * k
            dst = _PATCH * k
            for off in _VOFF:
                out_v[pl.ds(dst + off, 16)] = in_v[pl.ds(src + off, 16)]
            return 0

        lax.fori_loop(0, _K, patch, 0, unroll=2)
        pltpu.sync_copy(out_v, out_hbm.at[w, pl.ds(_PATCH * p0, _OUTLEN)])
        return 0

    lax.fori_loop(0, _NCHUNK, chunk, 0)


@jax.jit
def _extract_patches(x2):
    mesh = plsc.VectorSubcoreMesh(core_axis_name="c", subcore_axis_name="s")
    return pl.kernel(
        _sc_body,
        out_type=jax.ShapeDtypeStruct((_NW, _MAXP * _PATCH), jnp.float32),
        mesh=mesh,
        scratch_types=[
            pltpu.VMEM((_INLEN,), jnp.float32),
            pltpu.VMEM((_OUTLEN,), jnp.float32),
            pltpu.SemaphoreType.DMA,
        ],
        compiler_params=pltpu.CompilerParams(use_tc_tiling_on_sc=False),
    )(x2)


def kernel(x):
    B, C, T = x.shape
    assert (B * C, T) == (_NW, _T)
    x2 = x.reshape(_NW, _T)
    out = _extract_patches(x2)
    patches = out.reshape(B, C, _MAXP, _PATCH)
    masks = jnp.ones((B, C, _MAXP), jnp.float32)
    return (patches, masks)


# speedup vs baseline: 19.6074x; 1.0540x over previous
import functools

import jax
import jax.numpy as jnp
from jax import lax
from jax.experimental import pallas as pl
from jax.experimental.pallas import tpu as pltpu
from jax.experimental.pallas import tpu_sc as plsc

_PATCH = 200
_STRIDE = 150
_MAXP = 1000
_T = 150050
_NC, _NS = 2, 16
_NW = _NC * _NS
_K = 40
_NCHUNK = _MAXP // _K
_INLEN = _K * _STRIDE + 56
_OUTLEN = _K * _PATCH
_VOFF = tuple(16 * j for j in range(12)) + (184,)


def _sc_body(x_hbm, out_hbm, in_v, out_v, sem):
    w = lax.axis_index("s") * _NC + lax.axis_index("c")

    def chunk(ci, _):
        p0 = ci * _K
        pltpu.sync_copy(x_hbm.at[w, pl.ds(_STRIDE * p0, _INLEN)], in_v)

        for k in range(_K):
            src = _STRIDE * k
            dst = _PATCH * k
            for off in _VOFF:
                out_v[pl.ds(dst + off, 16)] = in_v[pl.ds(src + off, 16)]
        pltpu.sync_copy(out_v, out_hbm.at[w, pl.ds(_PATCH * p0, _OUTLEN)])
        return 0

    lax.fori_loop(0, _NCHUNK, chunk, 0)


@jax.jit
def _extract_patches(x2):
    mesh = plsc.VectorSubcoreMesh(core_axis_name="c", subcore_axis_name="s")
    return pl.kernel(
        _sc_body,
        out_type=jax.ShapeDtypeStruct((_NW, _MAXP * _PATCH), jnp.float32),
        mesh=mesh,
        scratch_types=[
            pltpu.VMEM((_INLEN,), jnp.float32),
            pltpu.VMEM((_OUTLEN,), jnp.float32),
            pltpu.SemaphoreType.DMA,
        ],
        compiler_params=pltpu.CompilerParams(use_tc_tiling_on_sc=False),
    )(x2)


def kernel(x):
    B, C, T = x.shape
    assert (B * C, T) == (_NW, _T)
    x2 = x.reshape(_NW, _T)
    out = _extract_patches(x2)
    patches = out.reshape(B, C, _MAXP, _PATCH)
    masks = jnp.ones((B, C, _MAXP), jnp.float32)
    return (patches, masks)
